# Initial kernel scaffold; baseline (speedup 1.0000x reference)
import jax
import jax.numpy as jnp
from jax.experimental import pallas as pl


def kernel(x, adj, Wr1, br1, Wl1, g1, be1, Wr2, br2, Wl2, g2, be2, Wr3, br3, Wl3, g3, be3, Wlin, blin):
    raise NotImplementedError("write your pallas kernel here")


# speedup vs baseline: 1.1305x; 1.1305x over previous
import jax
import jax.numpy as jnp
from jax.experimental import pallas as pl
from jax.experimental.pallas import tpu as pltpu

B, N, IN_C, HID, OUT_C = 16, 1024, 32, 32, 32
MTOT = float(B * N)
EPS = 1e-5


def _body(x_ref, adj_ref, wr1, br1, wl1, g1, be1, wr2, br2, wl2, g2, be2,
          wr3, br3, wl3, g3, be3, wlin, blin, out_ref,
          adjc, r_ref, s1, s2, a_ref, bb_ref):
    k = pl.program_id(0)
    b = pl.program_id(1)

    @pl.when(jnp.logical_and(k == 0, b == 0))
    def _init_stats():
        s1[...] = jnp.zeros_like(s1)
        s2[...] = jnp.zeros_like(s2)

    @pl.when(jnp.logical_and(k >= 1, b == 0))
    def _finalize_stats():
        g = jnp.where(k == 1, g1[...], jnp.where(k == 2, g2[...], g3[...]))
        be = jnp.where(k == 1, be1[...], jnp.where(k == 2, be2[...], be3[...]))
        mu = s1[...] / MTOT
        var = s2[...] / MTOT - mu * mu
        a = g * jax.lax.rsqrt(var + EPS)
        a_ref[k - 1] = a
        bb_ref[k - 1] = be - mu * a
        s1[...] = jnp.zeros_like(s1)
        s2[...] = jnp.zeros_like(s2)

    def layer(xin, agg, wr, brv, wl, j):
        conv = (jnp.dot(agg, wr[...], preferred_element_type=jnp.float32)
                + jnp.dot(xin, wl[...], preferred_element_type=jnp.float32)
                + brv[...])
        r = jnp.maximum(conv, 0.0)
        r_ref[j, b] = r.astype(jnp.bfloat16)
        s1[...] += jnp.sum(r, axis=0, keepdims=True)
        s2[...] += jnp.sum(r * r, axis=0, keepdims=True)

    @pl.when(k == 0)
    def _layer1():
        ab = adj_ref[0].astype(jnp.bfloat16)
        adjc[b] = ab
        xb = x_ref[0]
        agg = jnp.dot(ab, xb.astype(jnp.bfloat16),
                      preferred_element_type=jnp.float32)
        layer(xb, agg, wr1, br1, wl1, 0)

    @pl.when(k == 1)
    def _layer2():
        x1 = a_ref[0] * r_ref[0, b].astype(jnp.float32) + bb_ref[0]
        agg = jnp.dot(adjc[b], x1.astype(jnp.bfloat16),
                      preferred_element_type=jnp.float32)
        layer(x1, agg, wr2, br2, wl2, 1)

    @pl.when(k == 2)
    def _layer3():
        x2 = a_ref[1] * r_ref[1, b].astype(jnp.float32) + bb_ref[1]
        agg = jnp.dot(adjc[b], x2.astype(jnp.bfloat16),
                      preferred_element_type=jnp.float32)
        layer(x2, agg, wr3, br3, wl3, 2)

    @pl.when(k == 3)
    def _final():
        x1 = a_ref[0] * r_ref[0, b].astype(jnp.float32) + bb_ref[0]
        x2 = a_ref[1] * r_ref[1, b].astype(jnp.float32) + bb_ref[1]
        x3 = a_ref[2] * r_ref[2, b].astype(jnp.float32) + bb_ref[2]
        o = (jnp.dot(x1, wlin[0:HID], preferred_element_type=jnp.float32)
             + jnp.dot(x2, wlin[HID:2 * HID], preferred_element_type=jnp.float32)
             + jnp.dot(x3, wlin[2 * HID:], preferred_element_type=jnp.float32)
             + blin[...])
        out_ref[0] = jnp.maximum(o, 0.0)


def kernel(x, adj, Wr1, br1, Wl1, g1, be1, Wr2, br2, Wl2, g2, be2,
           Wr3, br3, Wl3, g3, be3, Wlin, blin):
    vec = lambda v: v.reshape(1, -1)

    def full(arr):
        nd = arr.ndim
        return pl.BlockSpec(arr.shape, lambda k, b: (0,) * nd)

    small = [vec(br1), Wl1, vec(g1), vec(be1),
             Wr2, vec(br2), Wl2, vec(g2), vec(be2),
             Wr3, vec(br3), Wl3, vec(g3), vec(be3),
             Wlin, vec(blin)]

    in_specs = (
        [pl.BlockSpec((1, N, IN_C),
                      lambda k, b: (jnp.where(k == 0, b, B - 1), 0, 0)),
         pl.BlockSpec((1, N, N),
                      lambda k, b: (jnp.where(k == 0, b, B - 1), 0, 0)),
         full(Wr1)]
        + [full(a) for a in small]
    )

    return pl.pallas_call(
        _body,
        grid=(4, B),
        in_specs=in_specs,
        out_specs=pl.BlockSpec((1, N, OUT_C),
                               lambda k, b: (jnp.where(k == 3, b, 0), 0, 0)),
        out_shape=jax.ShapeDtypeStruct((B, N, OUT_C), jnp.float32),
        scratch_shapes=[
            pltpu.VMEM((B, N, N), jnp.bfloat16),
            pltpu.VMEM((3, B, N, HID), jnp.bfloat16),
            pltpu.VMEM((1, HID), jnp.float32),
            pltpu.VMEM((1, HID), jnp.float32),
            pltpu.VMEM((3, 1, HID), jnp.float32),
            pltpu.VMEM((3, 1, HID), jnp.float32),
        ],
        compiler_params=pltpu.CompilerParams(
            dimension_semantics=("arbitrary", "arbitrary"),
            vmem_limit_bytes=112 * 1024 * 1024,
        ),
    )(x, adj, Wr1, *small)


# speedup vs baseline: 1.1560x; 1.0225x over previous
import jax
import jax.numpy as jnp
from jax.experimental import pallas as pl
from jax.experimental.pallas import tpu as pltpu

B, N, IN_C, HID, OUT_C = 16, 1024, 32, 32, 32
MTOT = float(B * N)
EPS = 1e-5


def _body(x_ref, adj_ref, wr1, br1, wl1, g1, be1, wr2, br2, wl2, g2, be2,
          wr3, br3, wl3, g3, be3, wlin, blin, out_ref,
          adjc, r_ref, s1, s2, a_ref, bb_ref):
    k = pl.program_id(0)
    b = pl.program_id(1)

    @pl.when(jnp.logical_and(k == 0, b == 0))
    def _init_stats():
        s1[...] = jnp.zeros_like(s1)
        s2[...] = jnp.zeros_like(s2)

    @pl.when(jnp.logical_and(k >= 1, b == 0))
    def _finalize_stats():
        g = jnp.where(k == 1, g1[...], jnp.where(k == 2, g2[...], g3[...]))
        be = jnp.where(k == 1, be1[...], jnp.where(k == 2, be2[...], be3[...]))
        mu = s1[...] / MTOT
        var = s2[...] / MTOT - mu * mu
        a = g * jax.lax.rsqrt(var + EPS)
        a_ref[k - 1] = a
        bb_ref[k - 1] = be - mu * a
        s1[...] = jnp.zeros_like(s1)
        s2[...] = jnp.zeros_like(s2)

    bf16 = jnp.bfloat16

    def layer(xin_bf, agg, wr, brv, wl, j):
        conv = (jnp.dot(agg.astype(bf16), wr[...].astype(bf16),
                        preferred_element_type=jnp.float32)
                + jnp.dot(xin_bf, wl[...].astype(bf16),
                          preferred_element_type=jnp.float32)
                + brv[...])
        r = jnp.maximum(conv, 0.0)
        r_ref[j, b] = r.astype(bf16)
        s1[...] += jnp.sum(r, axis=0, keepdims=True)
        s2[...] += jnp.sum(r * r, axis=0, keepdims=True)

    def bn_apply(j):
        xf = a_ref[j] * r_ref[j, b].astype(jnp.float32) + bb_ref[j]
        return xf.astype(bf16)

    @pl.when(k == 0)
    def _layer1():
        ab = adj_ref[0].astype(bf16)
        adjc[b] = ab
        xb = x_ref[0].astype(bf16)
        agg = jnp.dot(ab, xb, preferred_element_type=jnp.float32)
        layer(xb, agg, wr1, br1, wl1, 0)

    @pl.when(k == 1)
    def _layer2():
        x1 = bn_apply(0)
        agg = jnp.dot(adjc[b], x1, preferred_element_type=jnp.float32)
        layer(x1, agg, wr2, br2, wl2, 1)

    @pl.when(k == 2)
    def _layer3():
        x2 = bn_apply(1)
        agg = jnp.dot(adjc[b], x2, preferred_element_type=jnp.float32)
        layer(x2, agg, wr3, br3, wl3, 2)

    @pl.when(k == 3)
    def _final():
        o = (jnp.dot(bn_apply(0), wlin[0:HID].astype(bf16),
                     preferred_element_type=jnp.float32)
             + jnp.dot(bn_apply(1), wlin[HID:2 * HID].astype(bf16),
                       preferred_element_type=jnp.float32)
             + jnp.dot(bn_apply(2), wlin[2 * HID:].astype(bf16),
                       preferred_element_type=jnp.float32)
             + blin[...])
        out_ref[0] = jnp.maximum(o, 0.0)


def kernel(x, adj, Wr1, br1, Wl1, g1, be1, Wr2, br2, Wl2, g2, be2,
           Wr3, br3, Wl3, g3, be3, Wlin, blin):
    vec = lambda v: v.reshape(1, -1)

    def full(arr):
        nd = arr.ndim
        return pl.BlockSpec(arr.shape, lambda k, b: (0,) * nd)

    small = [vec(br1), Wl1, vec(g1), vec(be1),
             Wr2, vec(br2), Wl2, vec(g2), vec(be2),
             Wr3, vec(br3), Wl3, vec(g3), vec(be3),
             Wlin, vec(blin)]

    in_specs = (
        [pl.BlockSpec((1, N, IN_C),
                      lambda k, b: (jnp.where(k == 0, b, B - 1), 0, 0)),
         pl.BlockSpec((1, N, N),
                      lambda k, b: (jnp.where(k == 0, b, B - 1), 0, 0)),
         full(Wr1)]
        + [full(a) for a in small]
    )

    return pl.pallas_call(
        _body,
        grid=(4, B),
        in_specs=in_specs,
        out_specs=pl.BlockSpec((1, N, OUT_C),
                               lambda k, b: (jnp.where(k == 3, b, 0), 0, 0)),
        out_shape=jax.ShapeDtypeStruct((B, N, OUT_C), jnp.float32),
        scratch_shapes=[
            pltpu.VMEM((B, N, N), jnp.bfloat16),
            pltpu.VMEM((3, B, N, HID), jnp.bfloat16),
            pltpu.VMEM((1, HID), jnp.float32),
            pltpu.VMEM((1, HID), jnp.float32),
            pltpu.VMEM((3, 1, HID), jnp.float32),
            pltpu.VMEM((3, 1, HID), jnp.float32),
        ],
        compiler_params=pltpu.CompilerParams(
            dimension_semantics=("arbitrary", "arbitrary"),
            vmem_limit_bytes=112 * 1024 * 1024,
        ),
    )(x, adj, Wr1, *small)
